# baseline (speedup 1.0000x reference)
import functools

import jax
import jax.numpy as jnp
from jax import lax
from jax.experimental import pallas as pl
from jax.experimental.pallas import tpu as pltpu
from jax.experimental.pallas import tpu_sc as plsc

NUM_CORES = 2
NUM_SUBCORES = 16
NUM_WORKERS = NUM_CORES * NUM_SUBCORES

VOCAB = 1000000
B_BATCH = 4096
SEQ = 200
DIM = 64
PDIM = 128

TBLK = 128
N_FULL_BLOCKS = VOCAB // TBLK
TAIL = VOCAB - N_FULL_BLOCKS * TBLK
MAX_BLOCKS_PER_W = (N_FULL_BLOCKS + NUM_WORKERS - 1) // NUM_WORKERS

B_PER_W = B_BATCH // NUM_WORKERS
ROWS_PER_CHUNK = 2
CHUNK = ROWS_PER_CHUNK * SEQ
N_CHUNKS = B_PER_W // ROWS_PER_CHUNK
GATHER_SPLITS = ((0, 128), (128, 128), (256, 128), (384, 16))

_TILED = pltpu.CompilerParams(use_tc_tiling_on_sc=True)
_TILED_NOLAYOUT = pltpu.CompilerParams(
    use_tc_tiling_on_sc=True, needs_layout_passes=False
)


def _mesh():
    return plsc.VectorSubcoreMesh(core_axis_name="c", subcore_axis_name="s")


@jax.jit
def _build_table(wt, wtail):
    @functools.partial(
        pl.kernel,
        mesh=_mesh(),
        out_type=jax.ShapeDtypeStruct((VOCAB, PDIM), jnp.float32),
        scratch_types=[
            pltpu.VMEM((DIM, TBLK), jnp.float32),
            pltpu.VMEM((DIM, TBLK), jnp.float32),
            pltpu.VMEM((TBLK, PDIM), jnp.float32),
            pltpu.VMEM((TBLK, PDIM), jnp.float32),
            pltpu.SemaphoreType.DMA,
            pltpu.SemaphoreType.DMA,
            pltpu.SemaphoreType.DMA,
        ],
        compiler_params=_TILED_NOLAYOUT,
    )
    def k(wt_hbm, wtail_hbm, tab_hbm, tin0, tin1, tout0, tout1,
          sem_r, sem_w0, sem_w1):
        wid = lax.axis_index("s") * NUM_CORES + lax.axis_index("c")
        n_mine = (N_FULL_BLOCKS - wid + NUM_WORKERS - 1) // NUM_WORKERS
        iota = lax.iota(jnp.int32, 16)

        def read(tin, blk):
            return pltpu.make_async_copy(
                wt_hbm.at[:, pl.ds(blk * TBLK, TBLK)], tin, sem_r
            )

        def write(tout, sem_w, blk):
            return pltpu.make_async_copy(
                tout, tab_hbm.at[pl.ds(blk * TBLK, TBLK)], sem_w
            )

        def transpose(tin, tout):
            for g in range(TBLK // 16):
                rowg = 16 * g + iota
                for f in range(DIM):
                    x = tin[f, pl.ds(16 * g, 16)]
                    plsc.store_scatter(
                        tout, [rowg, jnp.full((16,), f, jnp.int32)], x
                    )

        bufs = ((tin0, tout0, sem_w0), (tin1, tout1, sem_w1))

        def step(i, slot):
            blk = wid + i * NUM_WORKERS

            @pl.when(i < n_mine)
            def _():
                tin, tout, sem_w = bufs[slot]
                if isinstance(i, int) and i < 2:
                    pass
                else:
                    @pl.when(i >= 2)
                    def _():
                        write(tout, sem_w, blk).wait()
                read(tin, blk).wait()

                @pl.when(i + 1 < n_mine)
                def _():
                    read(bufs[1 - slot][0], blk + NUM_WORKERS).start()

                transpose(tin, tout)
                write(tout, sem_w, blk).start()

        def body(i, carry):
            step(2 * i, 0)
            step(2 * i + 1, 1)
            return carry

        read(tin0, wid).start()
        step(0, 0)
        step(1, 1)
        lax.fori_loop(1, (MAX_BLOCKS_PER_W + 1) // 2, body, 0)
        write(tout0, sem_w0, wid).wait()
        write(tout1, sem_w1, wid).wait()

        @pl.when(wid == 0)
        def _():
            pltpu.sync_copy(wtail_hbm, tout0.at[pl.ds(0, TAIL)])
            pltpu.sync_copy(
                tout0.at[pl.ds(0, TAIL)],
                tab_hbm.at[pl.ds(VOCAB - TAIL, TAIL)],
            )

    return k(wt, wtail)


@jax.jit
def _embedding_gather(token_ids_flat, table):
    @functools.partial(
        pl.kernel,
        mesh=_mesh(),
        out_type=jax.ShapeDtypeStruct((B_BATCH, SEQ, PDIM), jnp.float32),
        scratch_types=[
            pltpu.VMEM((CHUNK,), jnp.int32),
            pltpu.VMEM((CHUNK,), jnp.int32),
            pltpu.VMEM((CHUNK, PDIM), jnp.float32),
            pltpu.VMEM((CHUNK, PDIM), jnp.float32),
            pltpu.SemaphoreType.DMA,
            pltpu.SemaphoreType.DMA,
            pltpu.SemaphoreType.DMA,
        ],
        compiler_params=_TILED,
    )
    def k(idx_hbm, table_hbm, out_hbm, idx0, idx1, rows0, rows1,
          sem_g, sem_w0, sem_w1):
        wid = lax.axis_index("s") * NUM_CORES + lax.axis_index("c")
        tok_base = wid * (B_PER_W * SEQ)
        b_base = wid * B_PER_W

        def gathers(idx_v, rows_v):
            return [
                pltpu.make_async_copy(
                    table_hbm.at[idx_v.at[pl.ds(g0, glen)]],
                    rows_v.at[pl.ds(g0, glen)],
                    sem_g,
                )
                for g0, glen in GATHER_SPLITS
            ]

        def writes(rows_v, sem_w, j):
            b0 = b_base + j * ROWS_PER_CHUNK
            return [
                pltpu.make_async_copy(
                    rows_v.at[pl.ds(r * SEQ, SEQ)],
                    out_hbm.at[b0 + r],
                    sem_w,
                )
                for r in range(ROWS_PER_CHUNK)
            ]

        def load_idx(idx_v, j):
            pltpu.sync_copy(
                idx_hbm.at[pl.ds(tok_base + j * CHUNK, CHUNK)], idx_v
            )

        bufs = ((idx0, rows0, sem_w0), (idx1, rows1, sem_w1))

        def do_chunk(j, slot, first2, last):
            idx_v, rows_v, sem_w = bufs[slot]
            idx_n = bufs[1 - slot][0]
            if not first2:
                for c in writes(rows_v, sem_w, j - 2):
                    c.wait()
            for c in gathers(idx_v, rows_v):
                c.start()
            if not last:
                load_idx(idx_n, j + 1)
            for c in gathers(idx_v, rows_v):
                c.wait()
            for c in writes(rows_v, sem_w, j):
                c.start()

        def body(i, carry):
            do_chunk(2 * i, 0, False, False)
            do_chunk(2 * i + 1, 1, False, False)
            return carry

        load_idx(idx0, 0)
        do_chunk(0, 0, True, False)
        do_chunk(1, 1, True, False)
        lax.fori_loop(1, N_CHUNKS // 2 - 1, body, 0)
        do_chunk(N_CHUNKS - 2, 0, False, False)
        do_chunk(N_CHUNKS - 1, 1, False, True)
        for c in writes(rows0, sem_w0, N_CHUNKS - 2):
            c.wait()
        for c in writes(rows1, sem_w1, N_CHUNKS - 1):
            c.wait()

    return k(token_ids_flat, table)


def kernel(token_ids, weight):
    s0, s1 = token_ids.shape
    flat = token_ids.reshape(s0 * s1)
    wt = weight.T
    wtail = jnp.pad(
        weight[N_FULL_BLOCKS * TBLK:], ((0, 0), (0, PDIM - DIM))
    )
    table = _build_table(wt, wtail)
    out = _embedding_gather(flat, table)
    return out[:, :, :DIM]


# speedup vs baseline: 1.2009x; 1.2009x over previous
import functools

import jax
import jax.numpy as jnp
from jax import lax
from jax.experimental import pallas as pl
from jax.experimental.pallas import tpu as pltpu
from jax.experimental.pallas import tpu_sc as plsc

NUM_CORES = 2
NUM_SUBCORES = 16
NUM_WORKERS = NUM_CORES * NUM_SUBCORES

VOCAB = 1000000
B_BATCH = 4096
SEQ = 200
DIM = 64
PDIM = 128

TBLK = 128
N_FULL_BLOCKS = VOCAB // TBLK
TAIL = VOCAB - N_FULL_BLOCKS * TBLK
MAX_BLOCKS_PER_W = (N_FULL_BLOCKS + NUM_WORKERS - 1) // NUM_WORKERS

B_PER_W = B_BATCH // NUM_WORKERS
ROWS_PER_CHUNK = 2
CHUNK = ROWS_PER_CHUNK * SEQ
N_CHUNKS = B_PER_W // ROWS_PER_CHUNK
GATHER_SPLITS = ((0, 128), (128, 128), (256, 128), (384, 16))

_TILED = pltpu.CompilerParams(use_tc_tiling_on_sc=True)
_TILED_NOLAYOUT = pltpu.CompilerParams(
    use_tc_tiling_on_sc=True, needs_layout_passes=False
)


def _mesh():
    return plsc.VectorSubcoreMesh(core_axis_name="c", subcore_axis_name="s")


@jax.jit
def _build_table(wt, wtail):
    @functools.partial(
        pl.kernel,
        mesh=_mesh(),
        out_type=jax.ShapeDtypeStruct((VOCAB, PDIM), jnp.float32),
        scratch_types=[
            pltpu.VMEM((DIM, TBLK), jnp.float32),
            pltpu.VMEM((DIM, TBLK), jnp.float32),
            pltpu.VMEM((TBLK, PDIM), jnp.float32),
            pltpu.VMEM((TBLK, PDIM), jnp.float32),
            pltpu.SemaphoreType.DMA,
            pltpu.SemaphoreType.DMA,
            pltpu.SemaphoreType.DMA,
        ],
        compiler_params=pltpu.CompilerParams(
            use_tc_tiling_on_sc=True, needs_layout_passes=False
        ),
    )
    def k(wt_hbm, wtail_hbm, tab_hbm, tin0, tin1, tout0, tout1,
          sem_r, sem_w0, sem_w1):
        wid = lax.axis_index("s") * NUM_CORES + lax.axis_index("c")
        n_mine = (N_FULL_BLOCKS - wid + NUM_WORKERS - 1) // NUM_WORKERS
        iota = lax.iota(jnp.int32, 16)

        def read(tin, blk):
            return pltpu.make_async_copy(
                wt_hbm.at[:, pl.ds(blk * TBLK, TBLK)], tin, sem_r
            )

        def write(tout, sem_w, blk):
            return pltpu.make_async_copy(
                tout, tab_hbm.at[pl.ds(blk * TBLK, TBLK)], sem_w
            )

        def transpose(tin, tout):
            for g in range(TBLK // 16):
                rowg = 16 * g + iota
                for f0 in range(0, DIM, 8):
                    xs = [tin[f0 + t, pl.ds(16 * g, 16)] for t in range(8)]
                    for t in range(8):
                        plsc.store_scatter(
                            tout,
                            [rowg, jnp.full((16,), f0 + t, jnp.int32)],
                            xs[t],
                        )

        bufs = ((tin0, tout0, sem_w0), (tin1, tout1, sem_w1))

        def step(i, slot):
            blk = wid + i * NUM_WORKERS

            @pl.when(i < n_mine)
            def _():
                tin, tout, sem_w = bufs[slot]
                if not (isinstance(i, int) and i < 2):
                    @pl.when(i >= 2)
                    def _():
                        write(tout, sem_w, blk).wait()
                read(tin, blk).wait()

                @pl.when(i + 1 < n_mine)
                def _():
                    read(bufs[1 - slot][0], blk + NUM_WORKERS).start()

                transpose(tin, tout)
                write(tout, sem_w, blk).start()

        def body(i, carry):
            step(2 * i, 0)
            step(2 * i + 1, 1)
            return carry

        read(tin0, wid).start()
        step(0, 0)
        step(1, 1)
        lax.fori_loop(1, (MAX_BLOCKS_PER_W + 1) // 2, body, 0)
        write(tout0, sem_w0, wid).wait()
        write(tout1, sem_w1, wid).wait()

        @pl.when(wid == 0)
        def _():
            pltpu.sync_copy(wtail_hbm, tout0.at[pl.ds(0, TAIL)])
            pltpu.sync_copy(
                tout0.at[pl.ds(0, TAIL)],
                tab_hbm.at[pl.ds(VOCAB - TAIL, TAIL)],
            )

    return k(wt, wtail)


@jax.jit
def _embedding_gather(token_ids_flat, table):
    @functools.partial(
        pl.kernel,
        mesh=_mesh(),
        out_type=jax.ShapeDtypeStruct((B_BATCH, SEQ, PDIM), jnp.float32),
        scratch_types=[
            pltpu.VMEM((CHUNK,), jnp.int32),
            pltpu.VMEM((CHUNK,), jnp.int32),
            pltpu.VMEM((CHUNK, PDIM), jnp.float32),
            pltpu.VMEM((CHUNK, PDIM), jnp.float32),
            pltpu.SemaphoreType.DMA,
            pltpu.SemaphoreType.DMA,
            pltpu.SemaphoreType.DMA,
        ],
        compiler_params=_TILED,
    )
    def k(idx_hbm, table_hbm, out_hbm, idx0, idx1, rows0, rows1,
          sem_g, sem_w0, sem_w1):
        wid = lax.axis_index("s") * NUM_CORES + lax.axis_index("c")
        tok_base = wid * (B_PER_W * SEQ)
        b_base = wid * B_PER_W

        def gathers(idx_v, rows_v):
            return [
                pltpu.make_async_copy(
                    table_hbm.at[idx_v.at[pl.ds(g0, glen)]],
                    rows_v.at[pl.ds(g0, glen)],
                    sem_g,
                )
                for g0, glen in GATHER_SPLITS
            ]

        def writes(rows_v, sem_w, j):
            b0 = b_base + j * ROWS_PER_CHUNK
            return [
                pltpu.make_async_copy(
                    rows_v.at[pl.ds(r * SEQ, SEQ)],
                    out_hbm.at[b0 + r],
                    sem_w,
                )
                for r in range(ROWS_PER_CHUNK)
            ]

        def load_idx(idx_v, j):
            pltpu.sync_copy(
                idx_hbm.at[pl.ds(tok_base + j * CHUNK, CHUNK)], idx_v
            )

        bufs = ((idx0, rows0, sem_w0), (idx1, rows1, sem_w1))

        def do_chunk(j, slot, first2, last):
            idx_v, rows_v, sem_w = bufs[slot]
            idx_n = bufs[1 - slot][0]
            if not first2:
                for c in writes(rows_v, sem_w, j - 2):
                    c.wait()
            for c in gathers(idx_v, rows_v):
                c.start()
            if not last:
                load_idx(idx_n, j + 1)
            for c in gathers(idx_v, rows_v):
                c.wait()
            for c in writes(rows_v, sem_w, j):
                c.start()

        def body(i, carry):
            do_chunk(2 * i, 0, False, False)
            do_chunk(2 * i + 1, 1, False, False)
            return carry

        load_idx(idx0, 0)
        do_chunk(0, 0, True, False)
        do_chunk(1, 1, True, False)
        lax.fori_loop(1, N_CHUNKS // 2 - 1, body, 0)
        do_chunk(N_CHUNKS - 2, 0, False, False)
        do_chunk(N_CHUNKS - 1, 1, False, True)
        for c in writes(rows0, sem_w0, N_CHUNKS - 2):
            c.wait()
        for c in writes(rows1, sem_w1, N_CHUNKS - 1):
            c.wait()

    return k(token_ids_flat, table)


def kernel(token_ids, weight):
    s0, s1 = token_ids.shape
    flat = token_ids.reshape(s0 * s1)
    wt = weight.T
    wtail = jnp.pad(
        weight[N_FULL_BLOCKS * TBLK:], ((0, 0), (0, PDIM - DIM))
    )
    table = _build_table(wt, wtail)
    out = _embedding_gather(flat, table)
    return out[:, :, :DIM]


# speedup vs baseline: 1.6842x; 1.4025x over previous
import functools

import jax
import jax.numpy as jnp
from jax import lax
from jax.experimental import pallas as pl
from jax.experimental.pallas import tpu as pltpu
from jax.experimental.pallas import tpu_sc as plsc

NUM_CORES = 2
NUM_SUBCORES = 16
NUM_WORKERS = NUM_CORES * NUM_SUBCORES

VOCAB = 1000000
B_BATCH = 4096
SEQ = 200
DIM = 64
PDIM = 128

TBLK = 128
N_FULL_BLOCKS = VOCAB // TBLK
TAIL = VOCAB - N_FULL_BLOCKS * TBLK
MAX_BLOCKS_PER_W = (N_FULL_BLOCKS + NUM_WORKERS - 1) // NUM_WORKERS

B_PER_W = B_BATCH // NUM_WORKERS
ROWS_PER_CHUNK = 2
CHUNK = ROWS_PER_CHUNK * SEQ
N_CHUNKS = B_PER_W // ROWS_PER_CHUNK
GATHER_SPLITS = ((0, 128), (128, 128), (256, 128), (384, 16))

_TILED = pltpu.CompilerParams(use_tc_tiling_on_sc=True)
_TILED_NOLAYOUT = pltpu.CompilerParams(
    use_tc_tiling_on_sc=True, needs_layout_passes=False
)


def _mesh():
    return plsc.VectorSubcoreMesh(core_axis_name="c", subcore_axis_name="s")


@jax.jit
def _embedding_gather(token_ids_flat, table):
    @functools.partial(
        pl.kernel,
        mesh=_mesh(),
        out_type=jax.ShapeDtypeStruct((B_BATCH, SEQ, PDIM), jnp.float32),
        scratch_types=[
            pltpu.VMEM((CHUNK,), jnp.int32),
            pltpu.VMEM((CHUNK,), jnp.int32),
            pltpu.VMEM((CHUNK, PDIM), jnp.float32),
            pltpu.VMEM((CHUNK, PDIM), jnp.float32),
            pltpu.SemaphoreType.DMA,
            pltpu.SemaphoreType.DMA,
            pltpu.SemaphoreType.DMA,
        ],
        compiler_params=_TILED,
    )
    def k(idx_hbm, table_hbm, out_hbm, idx0, idx1, rows0, rows1,
          sem_g, sem_w0, sem_w1):
        wid = lax.axis_index("s") * NUM_CORES + lax.axis_index("c")
        tok_base = wid * (B_PER_W * SEQ)
        b_base = wid * B_PER_W

        def gathers(idx_v, rows_v):
            return [
                pltpu.make_async_copy(
                    table_hbm.at[idx_v.at[pl.ds(g0, glen)]],
                    rows_v.at[pl.ds(g0, glen)],
                    sem_g,
                )
                for g0, glen in GATHER_SPLITS
            ]

        def writes(rows_v, sem_w, j):
            b0 = b_base + j * ROWS_PER_CHUNK
            return [
                pltpu.make_async_copy(
                    rows_v.at[pl.ds(r * SEQ, SEQ)],
                    out_hbm.at[b0 + r],
                    sem_w,
                )
                for r in range(ROWS_PER_CHUNK)
            ]

        def load_idx(idx_v, j):
            pltpu.sync_copy(
                idx_hbm.at[pl.ds(tok_base + j * CHUNK, CHUNK)], idx_v
            )

        bufs = ((idx0, rows0, sem_w0), (idx1, rows1, sem_w1))

        def do_chunk(j, slot, first2, last):
            idx_v, rows_v, sem_w = bufs[slot]
            idx_n = bufs[1 - slot][0]
            if not first2:
                for c in writes(rows_v, sem_w, j - 2):
                    c.wait()
            for c in gathers(idx_v, rows_v):
                c.start()
            if not last:
                load_idx(idx_n, j + 1)
            for c in gathers(idx_v, rows_v):
                c.wait()
            for c in writes(rows_v, sem_w, j):
                c.start()

        def body(i, carry):
            do_chunk(2 * i, 0, False, False)
            do_chunk(2 * i + 1, 1, False, False)
            return carry

        load_idx(idx0, 0)
        do_chunk(0, 0, True, False)
        do_chunk(1, 1, True, False)
        lax.fori_loop(1, N_CHUNKS // 2 - 1, body, 0)
        do_chunk(N_CHUNKS - 2, 0, False, False)
        do_chunk(N_CHUNKS - 1, 1, False, True)
        for c in writes(rows0, sem_w0, N_CHUNKS - 2):
            c.wait()
        for c in writes(rows1, sem_w1, N_CHUNKS - 1):
            c.wait()

    return k(token_ids_flat, table)


def kernel(token_ids, weight):
    s0, s1 = token_ids.shape
    flat = token_ids.reshape(s0 * s1)
    table = jnp.pad(weight, ((0, 0), (0, PDIM - DIM)))
    out = _embedding_gather(flat, table)
    return out[:, :, :DIM]
